# baseline (speedup 1.0000x reference)
import functools
import math

import jax
import jax.numpy as jnp
from jax import lax
from jax.experimental import pallas as pl
from jax.experimental.pallas import tpu as pltpu
from jax.experimental.pallas import tpu_sc as plsc

N_BINS = 500
K_FINE = 4
FINE = N_BINS * K_FINE
FINE_PAD = 2048
NC = 2
NS = 16
L = 16
NW = NC * NS
N_ELEM = 16777216
PER_W = N_ELEM // NW
CHUNK = 16384
NCHUNK = PER_W // CHUNK
MM_CHUNK = 32768
MM_NCHUNK = PER_W // MM_CHUNK
HIST_WORDS = 2 * FINE_PAD * L
NEG_HALF_LOG_2PI = -0.5 * math.log(2.0 * math.pi)

_mesh = plsc.VectorSubcoreMesh(
    core_axis_name="c", subcore_axis_name="s", num_cores=NC, num_subcores=NS
)


def _wid():
    return lax.axis_index("s") * NC + lax.axis_index("c")


def _splat(val):
    return jnp.where(lax.iota(jnp.int32, L) >= 0, jnp.float32(val), jnp.float32(0))


def _ring_scan(x_hbm, base, buf0, buf1, sem0, sem1, compute, init, chunk, nchunk):
    def start(c, buf, sem):
        s = pl.multiple_of(base + c * chunk, chunk)
        pltpu.make_async_copy(x_hbm.at[pl.ds(s, chunk)], buf, sem).start()

    def wait(buf, sem):
        pltpu.make_async_copy(x_hbm.at[pl.ds(0, chunk)], buf, sem).wait()

    start(0, buf0, sem0)
    start(1, buf1, sem1)

    def body(k, carry):
        wait(buf0, sem0)
        carry = compute(buf0, carry)

        @pl.when(2 * k + 2 < nchunk)
        def _s0():
            start(2 * k + 2, buf0, sem0)

        wait(buf1, sem1)
        carry = compute(buf1, carry)

        @pl.when(2 * k + 3 < nchunk)
        def _s1():
            start(2 * k + 3, buf1, sem1)

        return carry

    return lax.fori_loop(0, nchunk // 2, body, init)


@functools.partial(
    pl.kernel,
    out_type=jax.ShapeDtypeStruct((NW * 32,), jnp.float32),
    mesh=_mesh,
    scratch_types=[
        pltpu.VMEM((MM_CHUNK,), jnp.float32),
        pltpu.VMEM((MM_CHUNK,), jnp.float32),
        pltpu.VMEM((32,), jnp.float32),
        pltpu.SemaphoreType.DMA,
        pltpu.SemaphoreType.DMA,
    ],
)
def _minmax_sc_kernel(xa_hbm, out_hbm, buf0, buf1, mmv, sem0, sem1):
    wid = _wid()
    base = wid * PER_W

    big = _splat(jnp.inf)
    nacc = 8
    nvec8 = MM_CHUNK // L // nacc

    def compute(buf, carry):
        def body8(i, c2):
            mns, mxs = c2
            mns, mxs = list(mns), list(mxs)
            for u in range(nacc):
                v = buf[pl.ds((i * nacc + u) * L, L)]
                mns[u] = jnp.minimum(mns[u], v)
                mxs[u] = jnp.maximum(mxs[u], v)
            return tuple(mns), tuple(mxs)

        return lax.fori_loop(0, nvec8, body8, carry)

    mns, mxs = _ring_scan(
        xa_hbm, base, buf0, buf1, sem0, sem1, compute,
        ((big,) * nacc, (-big,) * nacc), MM_CHUNK, MM_NCHUNK,
    )
    mmv[pl.ds(0, L)] = functools.reduce(jnp.minimum, mns)
    mmv[pl.ds(16, L)] = functools.reduce(jnp.maximum, mxs)
    pltpu.sync_copy(mmv, out_hbm.at[pl.ds(wid * 32, 32)])


_TC_ROWS = 8192
_TC_COLS = N_ELEM // _TC_ROWS
_TC_GRID = 16
_TC_BLK = _TC_ROWS // _TC_GRID


def _minmax_tc_body(x_ref, o_ref, acc_mn, acc_mx):
    i = pl.program_id(0)
    x = x_ref[...].reshape(8, _TC_BLK // 8, _TC_COLS)
    mn = x.min(axis=1)
    mx = x.max(axis=1)

    @pl.when(i == 0)
    def _init():
        acc_mn[...] = mn
        acc_mx[...] = mx

    @pl.when(i > 0)
    def _acc():
        acc_mn[...] = jnp.minimum(acc_mn[...], mn)
        acc_mx[...] = jnp.maximum(acc_mx[...], mx)

    @pl.when(i == _TC_GRID - 1)
    def _emit():
        mn_s = jnp.min(acc_mn[...])
        mx_s = jnp.max(acc_mx[...])
        o_ref[...] = jnp.concatenate(
            [jnp.full((1, 128), mn_s, jnp.float32),
             jnp.full((1, 128), mx_s, jnp.float32)],
            axis=0,
        )


def _minmax_tc(x2d):
    return pl.pallas_call(
        _minmax_tc_body,
        grid=(_TC_GRID,),
        in_specs=[pl.BlockSpec((_TC_BLK, _TC_COLS), lambda i: (i, 0))],
        out_specs=pl.BlockSpec((2, 128), lambda i: (0, 0)),
        out_shape=jax.ShapeDtypeStruct((2, 128), jnp.float32),
        scratch_shapes=[
            pltpu.VMEM((8, _TC_COLS), jnp.float32),
            pltpu.VMEM((8, _TC_COLS), jnp.float32),
        ],
    )(x2d)


@functools.partial(
    pl.kernel,
    out_type=jax.ShapeDtypeStruct((NW * HIST_WORDS,), jnp.float32),
    mesh=_mesh,
    scratch_types=[
        pltpu.VMEM((CHUNK,), jnp.float32),
        pltpu.VMEM((CHUNK,), jnp.float32),
        pltpu.VMEM((HIST_WORDS,), jnp.float32),
        pltpu.VMEM((NW * 32,), jnp.float32),
        pltpu.VMEM((256,), jnp.float32),
        pltpu.SemaphoreType.DMA,
        pltpu.SemaphoreType.DMA,
    ],
    compiler_params=pltpu.CompilerParams(needs_layout_passes=False),
)
def _hist_kernel(
    xa_hbm, xb_hbm, mmsc_hbm, mmtc_hbm, out_hbm,
    buf0, buf1, hist, mmv, mmtv, sem0, sem1,
):
    def lane_reduce(v, op):
        s = v[0]
        for i in range(1, L):
            s = op(s, v[i])
        return jnp.full((L,), s, jnp.float32)

    wid = _wid()
    base = wid * PER_W

    pltpu.sync_copy(mmsc_hbm, mmv)
    pltpu.sync_copy(mmtc_hbm, mmtv)

    big = _splat(jnp.inf)

    def fold_body(w, carry):
        mna, mxa = carry
        o = w * 32
        return (
            jnp.minimum(mna, mmv[pl.ds(o, L)]),
            jnp.maximum(mxa, mmv[pl.ds(o + 16, L)]),
        )

    mna, mxa = lax.fori_loop(0, NW, fold_body, (big, -big))

    fine_f = jnp.float32(FINE)
    one = _splat(1.0)

    lo_a = lane_reduce(mna, jnp.minimum)
    hi_a = lane_reduce(mxa, jnp.maximum)
    lo_b = mmtv[pl.ds(0, L)]
    hi_b = mmtv[pl.ds(128, L)]
    inv_a = one / ((hi_a - lo_a) / fine_f)
    inv_b = one / ((hi_b - lo_b) / fine_f)

    zeros = _splat(0.0)

    def zero_body(i, _):
        hist[pl.ds(i * L, L)] = zeros
        return 0

    lax.fori_loop(0, HIST_WORDS // L, zero_body, 0)

    lane = lax.iota(jnp.int32, L)
    fine_pad_f = jnp.float32(FINE_PAD)

    def scan_array(x_hbm, lo_own, inv_own, own_off):
        own_lanes = lane + own_off
        off_own = fine_pad_f - lo_own * inv_own

        def compute(buf, carry):
            def body(j):
                v = buf[pl.ds(j * L, L)]
                t = v * inv_own + off_own
                a = plsc.bitcast(t, jnp.int32) >> 8
                plsc.addupdate_scatter(hist, [(a & 0x7FF0) | own_lanes], one)

            plsc.parallel_loop(0, CHUNK // L, unroll=8)(body)
            return carry

        _ring_scan(x_hbm, base, buf0, buf1, sem0, sem1, compute, 0, CHUNK, NCHUNK)

    scan_array(xa_hbm, lo_a, inv_a, 0)
    scan_array(xb_hbm, lo_b, inv_b, FINE_PAD * L)

    pltpu.sync_copy(hist, out_hbm.at[pl.ds(wid * HIST_WORDS, HIST_WORDS)])


_ROWS_PER_HIST = FINE_PAD * L // 128
_GRP = 128 // L


def _finalize_body(h_ref, mm_ref, mmt_ref, o_ref):
    mm = mm_ref[...]
    mmt = mmt_ref[...]
    lo_a = jnp.min(mm[:, 0:16])
    hi_a = jnp.max(mm[:, 16:32])
    lo_b = jnp.min(mmt[0:1, :])
    hi_b = jnp.max(mmt[1:2, :])
    lo_ab = jnp.minimum(lo_a, lo_b)
    hi_ab = jnp.maximum(hi_a, hi_b)
    w_ab = (hi_ab - lo_ab) / N_BINS
    inv_ab = 1.0 / w_ab

    h = h_ref[...].reshape(NW, 2 * _ROWS_PER_HIST, 128).sum(axis=0)

    sel = (
        lax.broadcasted_iota(jnp.int32, (128, _GRP), 0) // L
        == lax.broadcasted_iota(jnp.int32, (128, _GRP), 1)
    ).astype(jnp.float32)

    fshape = (_ROWS_PER_HIST, _GRP)
    f_idx = (
        lax.broadcasted_iota(jnp.int32, fshape, 0) * _GRP
        + lax.broadcasted_iota(jnp.int32, fshape, 1)
    ).astype(jnp.float32)

    def lp_ab(j):
        c = lo_ab + (j + 0.5) * w_ab
        return -0.5 * c * c + NEG_HALF_LOG_2PI

    own_terms = []
    ab_parts = []
    totals = []
    for hist_i, (lo, hi) in enumerate([(lo_a, hi_a), (lo_b, hi_b)]):
        block = h[hist_i * _ROWS_PER_HIST : (hist_i + 1) * _ROWS_PER_HIST, :]
        fine_mat = jnp.dot(block, sel, preferred_element_type=jnp.float32)

        w_own = (hi - lo) / N_BINS
        ob = jnp.minimum(jnp.floor(f_idx * (1.0 / K_FINE)), N_BINS - 1.0)
        c_own = lo + (ob + 0.5) * w_own
        lp_own = -0.5 * c_own * c_own + NEG_HALF_LOG_2PI
        tot = jnp.sum(fine_mat)
        own_terms.append(jnp.sum(fine_mat * lp_own) / tot)
        totals.append(tot)

        w_fine = (hi - lo) / FINE
        u0 = lo + f_idx * w_fine
        u1 = u0 + w_fine
        j0 = jnp.clip(jnp.floor((u0 - lo_ab) * inv_ab), 0.0, N_BINS - 1.0)
        j1 = jnp.clip(jnp.floor((u1 - lo_ab) * inv_ab), 0.0, N_BINS - 1.0)
        t = jnp.clip((u1 - (lo_ab + j1 * w_ab)) / w_fine, 0.0, 1.0)
        g = lp_ab(j0) * (1.0 - t) + lp_ab(j1) * t
        ab_parts.append(jnp.sum(fine_mat * g))

    avg_ab = (ab_parts[0] + ab_parts[1]) / (totals[0] + totals[1])
    log_s = avg_ab - own_terms[0] - own_terms[1]
    o_ref[...] = jnp.reshape(log_s, (1, 1))


def kernel(XA_1d, XB_1d):
    mm_sc = _minmax_sc_kernel(XA_1d)
    mm_tc = _minmax_tc(XB_1d.reshape(_TC_ROWS, _TC_COLS))
    hists = _hist_kernel(XA_1d, XB_1d, mm_sc, mm_tc.reshape(256))
    out = pl.pallas_call(
        _finalize_body,
        out_shape=jax.ShapeDtypeStruct((1, 1), jnp.float32),
    )(
        hists.reshape(NW * 2 * _ROWS_PER_HIST, 128),
        mm_sc.reshape(NW, 32),
        mm_tc,
    )
    return out[0, 0]


# speedup vs baseline: 1.2013x; 1.2013x over previous
import functools
import math

import jax
import jax.numpy as jnp
from jax import lax
from jax.experimental import pallas as pl
from jax.experimental.pallas import tpu as pltpu
from jax.experimental.pallas import tpu_sc as plsc

N_BINS = 500
K_FINE = 4
FINE = N_BINS * K_FINE
FINE_PAD = 2048
NC = 2
NS = 16
L = 16
NW = NC * NS
N_ELEM = 16777216
PER_W = N_ELEM // NW
CHUNK = 16384
NCHUNK = PER_W // CHUNK
MM_CHUNK = 32768
MM_NCHUNK = PER_W // MM_CHUNK
HIST_WORDS = 2 * FINE_PAD * L
NEG_HALF_LOG_2PI = -0.5 * math.log(2.0 * math.pi)

_mesh = plsc.VectorSubcoreMesh(
    core_axis_name="c", subcore_axis_name="s", num_cores=NC, num_subcores=NS
)


def _wid():
    return lax.axis_index("s") * NC + lax.axis_index("c")


def _splat(val):
    return jnp.where(lax.iota(jnp.int32, L) >= 0, jnp.float32(val), jnp.float32(0))


def _ring_scan(x_hbm, base, buf0, buf1, sem0, sem1, compute, init, chunk, nchunk):
    def start(c, buf, sem):
        s = pl.multiple_of(base + c * chunk, chunk)
        pltpu.make_async_copy(x_hbm.at[pl.ds(s, chunk)], buf, sem).start()

    def wait(buf, sem):
        pltpu.make_async_copy(x_hbm.at[pl.ds(0, chunk)], buf, sem).wait()

    start(0, buf0, sem0)
    start(1, buf1, sem1)

    def body(k, carry):
        wait(buf0, sem0)
        carry = compute(buf0, carry)

        @pl.when(2 * k + 2 < nchunk)
        def _s0():
            start(2 * k + 2, buf0, sem0)

        wait(buf1, sem1)
        carry = compute(buf1, carry)

        @pl.when(2 * k + 3 < nchunk)
        def _s1():
            start(2 * k + 3, buf1, sem1)

        return carry

    return lax.fori_loop(0, nchunk // 2, body, init)


@functools.partial(
    pl.kernel,
    out_type=jax.ShapeDtypeStruct((NW * 64,), jnp.float32),
    mesh=_mesh,
    scratch_types=[
        pltpu.VMEM((MM_CHUNK,), jnp.float32),
        pltpu.VMEM((MM_CHUNK,), jnp.float32),
        pltpu.VMEM((64,), jnp.float32),
        pltpu.SemaphoreType.DMA,
        pltpu.SemaphoreType.DMA,
    ],
)
def _minmax_sc_kernel(xa_hbm, xb_hbm, out_hbm, buf0, buf1, mmv, sem0, sem1):
    wid = _wid()
    base = wid * PER_W

    big = _splat(jnp.inf)
    nacc = 8
    nvec8 = MM_CHUNK // L // nacc

    def scan_array(x_hbm):
        def compute(buf, carry):
            def body8(i, c2):
                mns, mxs = c2
                mns, mxs = list(mns), list(mxs)
                for u in range(nacc):
                    v = buf[pl.ds((i * nacc + u) * L, L)]
                    mns[u] = jnp.minimum(mns[u], v)
                    mxs[u] = jnp.maximum(mxs[u], v)
                return tuple(mns), tuple(mxs)

            return lax.fori_loop(0, nvec8, body8, carry)

        mns, mxs = _ring_scan(
            x_hbm, base, buf0, buf1, sem0, sem1, compute,
            ((big,) * nacc, (-big,) * nacc), MM_CHUNK, MM_NCHUNK,
        )
        return (
            functools.reduce(jnp.minimum, mns),
            functools.reduce(jnp.maximum, mxs),
        )

    mna, mxa = scan_array(xa_hbm)
    mnb, mxb = scan_array(xb_hbm)
    mmv[pl.ds(0, L)] = mna
    mmv[pl.ds(16, L)] = mxa
    mmv[pl.ds(32, L)] = mnb
    mmv[pl.ds(48, L)] = mxb
    pltpu.sync_copy(mmv, out_hbm.at[pl.ds(wid * 64, 64)])


@functools.partial(
    pl.kernel,
    out_type=jax.ShapeDtypeStruct((NW * HIST_WORDS,), jnp.float32),
    mesh=_mesh,
    scratch_types=[
        pltpu.VMEM((CHUNK,), jnp.float32),
        pltpu.VMEM((CHUNK,), jnp.float32),
        pltpu.VMEM((HIST_WORDS,), jnp.float32),
        pltpu.VMEM((NW * 64,), jnp.float32),
        pltpu.SemaphoreType.DMA,
        pltpu.SemaphoreType.DMA,
    ],
    compiler_params=pltpu.CompilerParams(needs_layout_passes=False),
)
def _hist_kernel(
    xa_hbm, xb_hbm, mm_hbm, out_hbm, buf0, buf1, hist, mmv, sem0, sem1
):
    def lane_reduce(v, op):
        s = v[0]
        for i in range(1, L):
            s = op(s, v[i])
        return jnp.full((L,), s, jnp.float32)

    wid = _wid()
    base = wid * PER_W

    pltpu.sync_copy(mm_hbm, mmv)

    big = _splat(jnp.inf)

    def fold_body(w, carry):
        mna, mxa, mnb, mxb = carry
        o = w * 64
        return (
            jnp.minimum(mna, mmv[pl.ds(o, L)]),
            jnp.maximum(mxa, mmv[pl.ds(o + 16, L)]),
            jnp.minimum(mnb, mmv[pl.ds(o + 32, L)]),
            jnp.maximum(mxb, mmv[pl.ds(o + 48, L)]),
        )

    mna, mxa, mnb, mxb = lax.fori_loop(0, NW, fold_body, (big, -big, big, -big))

    fine_f = jnp.float32(FINE)
    one = _splat(1.0)

    lo_a = lane_reduce(mna, jnp.minimum)
    hi_a = lane_reduce(mxa, jnp.maximum)
    lo_b = lane_reduce(mnb, jnp.minimum)
    hi_b = lane_reduce(mxb, jnp.maximum)
    inv_a = one / ((hi_a - lo_a) / fine_f)
    inv_b = one / ((hi_b - lo_b) / fine_f)

    zeros = _splat(0.0)

    def zero_body(i, _):
        hist[pl.ds(i * L, L)] = zeros
        return 0

    lax.fori_loop(0, HIST_WORDS // L, zero_body, 0)

    lane = lax.iota(jnp.int32, L)
    fine_pad_f = jnp.float32(FINE_PAD)

    def scan_array(x_hbm, lo_own, inv_own, own_off):
        own_lanes = lane + own_off
        off_own = fine_pad_f - lo_own * inv_own

        def compute(buf, carry):
            def body(j):
                v = buf[pl.ds(j * L, L)]
                t = v * inv_own + off_own
                a = plsc.bitcast(t, jnp.int32) >> 8
                plsc.addupdate_scatter(hist, [(a & 0x7FF0) | own_lanes], one)

            plsc.parallel_loop(0, CHUNK // L, unroll=8)(body)
            return carry

        _ring_scan(x_hbm, base, buf0, buf1, sem0, sem1, compute, 0, CHUNK, NCHUNK)

    scan_array(xa_hbm, lo_a, inv_a, 0)
    scan_array(xb_hbm, lo_b, inv_b, FINE_PAD * L)

    pltpu.sync_copy(hist, out_hbm.at[pl.ds(wid * HIST_WORDS, HIST_WORDS)])


_ROWS_PER_HIST = FINE_PAD * L // 128
_GRP = 128 // L


def _finalize_body(h_ref, mm_ref, o_ref):
    mm = mm_ref[...]
    lo_a = jnp.min(mm[:, 0:16])
    hi_a = jnp.max(mm[:, 16:32])
    lo_b = jnp.min(mm[:, 32:48])
    hi_b = jnp.max(mm[:, 48:64])
    lo_ab = jnp.minimum(lo_a, lo_b)
    hi_ab = jnp.maximum(hi_a, hi_b)
    w_ab = (hi_ab - lo_ab) / N_BINS
    inv_ab = 1.0 / w_ab

    h = h_ref[...].reshape(NW, 2 * _ROWS_PER_HIST, 128).sum(axis=0)

    sel = (
        lax.broadcasted_iota(jnp.int32, (128, _GRP), 0) // L
        == lax.broadcasted_iota(jnp.int32, (128, _GRP), 1)
    ).astype(jnp.float32)

    fshape = (_ROWS_PER_HIST, _GRP)
    f_idx = (
        lax.broadcasted_iota(jnp.int32, fshape, 0) * _GRP
        + lax.broadcasted_iota(jnp.int32, fshape, 1)
    ).astype(jnp.float32)

    def lp_ab(j):
        c = lo_ab + (j + 0.5) * w_ab
        return -0.5 * c * c + NEG_HALF_LOG_2PI

    own_terms = []
    ab_parts = []
    totals = []
    for hist_i, (lo, hi) in enumerate([(lo_a, hi_a), (lo_b, hi_b)]):
        block = h[hist_i * _ROWS_PER_HIST : (hist_i + 1) * _ROWS_PER_HIST, :]
        fine_mat = jnp.dot(block, sel, preferred_element_type=jnp.float32)

        w_own = (hi - lo) / N_BINS
        ob = jnp.minimum(jnp.floor(f_idx * (1.0 / K_FINE)), N_BINS - 1.0)
        c_own = lo + (ob + 0.5) * w_own
        lp_own = -0.5 * c_own * c_own + NEG_HALF_LOG_2PI
        tot = jnp.sum(fine_mat)
        own_terms.append(jnp.sum(fine_mat * lp_own) / tot)
        totals.append(tot)

        w_fine = (hi - lo) / FINE
        u0 = lo + f_idx * w_fine
        u1 = u0 + w_fine
        j0 = jnp.clip(jnp.floor((u0 - lo_ab) * inv_ab), 0.0, N_BINS - 1.0)
        j1 = jnp.clip(jnp.floor((u1 - lo_ab) * inv_ab), 0.0, N_BINS - 1.0)
        t = jnp.clip((u1 - (lo_ab + j1 * w_ab)) / w_fine, 0.0, 1.0)
        g = lp_ab(j0) * (1.0 - t) + lp_ab(j1) * t
        ab_parts.append(jnp.sum(fine_mat * g))

    avg_ab = (ab_parts[0] + ab_parts[1]) / (totals[0] + totals[1])
    log_s = avg_ab - own_terms[0] - own_terms[1]
    o_ref[...] = jnp.reshape(log_s, (1, 1))


def kernel(XA_1d, XB_1d):
    mm = _minmax_sc_kernel(XA_1d, XB_1d)
    hists = _hist_kernel(XA_1d, XB_1d, mm)
    out = pl.pallas_call(
        _finalize_body,
        out_shape=jax.ShapeDtypeStruct((1, 1), jnp.float32),
    )(hists.reshape(NW * 2 * _ROWS_PER_HIST, 128), mm.reshape(NW, 64))
    return out[0, 0]
